# Initial kernel scaffold; baseline (speedup 1.0000x reference)
import functools

import jax
import jax.numpy as jnp
from jax import lax
from jax.experimental import pallas as pl
from jax.experimental.pallas import tpu as pltpu
from jax.experimental.pallas import tpu_sc as plsc

NUM_CORES = 2
NUM_SUBCORES = 16
NUM_WORKERS = NUM_CORES * NUM_SUBCORES

BATCH = 16384
PER_TEAM = 50
EMBED_DIM = 64
TOTAL = BATCH * PER_TEAM
ROWS_PER_WORKER = TOTAL // NUM_WORKERS
CHUNK = 128
NCHUNKS = ROWS_PER_WORKER // CHUNK


def _gather_kernel(table_hbm, idx_hbm, out_hbm, idx_v, rows_v, gsem, wsem):
    wid = lax.axis_index("s") * NUM_CORES + lax.axis_index("c")
    base = wid * ROWS_PER_WORKER

    pltpu.sync_copy(idx_hbm.at[wid], idx_v)

    def body(c, carry):
        pltpu.async_copy(table_hbm.at[idx_v.at[c]], rows_v, gsem).wait()
        pltpu.async_copy(rows_v, out_hbm.at[pl.ds(base + c * CHUNK, CHUNK)],
                         wsem).wait()
        return carry

    lax.fori_loop(0, NCHUNKS, body, 0, unroll=False)


@jax.jit
def _embed(ids_flat, table):
    mesh = plsc.VectorSubcoreMesh(core_axis_name="c", subcore_axis_name="s")
    run = pl.kernel(
        _gather_kernel,
        out_type=jax.ShapeDtypeStruct((TOTAL, EMBED_DIM), jnp.float32),
        mesh=mesh,
        scratch_types=[
            pltpu.VMEM((NCHUNKS, CHUNK), jnp.int32),
            pltpu.VMEM((CHUNK, EMBED_DIM), jnp.float32),
            pltpu.SemaphoreType.DMA,
            pltpu.SemaphoreType.DMA,
        ],
    )
    idx = ids_flat.reshape(NUM_WORKERS, NCHUNKS, CHUNK).astype(jnp.int32)
    return run(table, idx)


def kernel(champion_ids, table):
    out = _embed(champion_ids.reshape(-1), table)
    return out.reshape(BATCH, PER_TEAM, EMBED_DIM)


# speedup vs baseline: 5.2140x; 5.2140x over previous
import functools

import jax
import jax.numpy as jnp
from jax import lax
from jax.experimental import pallas as pl
from jax.experimental.pallas import tpu as pltpu
from jax.experimental.pallas import tpu_sc as plsc

NUM_CORES = 2
NUM_SUBCORES = 16
NUM_WORKERS = NUM_CORES * NUM_SUBCORES

BATCH = 16384
PER_TEAM = 50
EMBED_DIM = 64
TOTAL = BATCH * PER_TEAM
ROWS_PER_WORKER = TOTAL // NUM_WORKERS
CHUNK = 128
NCHUNKS = ROWS_PER_WORKER // CHUNK


def _gather_kernel(table_hbm, idx_hbm, out_hbm, idx_v, rows_v, gsem, wsem):
    wid = lax.axis_index("s") * NUM_CORES + lax.axis_index("c")
    base = wid * ROWS_PER_WORKER

    pltpu.sync_copy(idx_hbm.at[wid], idx_v)

    def body(c, carry):
        pltpu.async_copy(table_hbm.at[idx_v.at[c]], rows_v, gsem).wait()
        pltpu.async_copy(rows_v, out_hbm.at[pl.ds(base + c * CHUNK, CHUNK)],
                         wsem).wait()
        return carry

    lax.fori_loop(0, NCHUNKS, body, 0, unroll=False)


@jax.jit
def _embed(ids_flat, table):
    mesh = plsc.VectorSubcoreMesh(core_axis_name="c", subcore_axis_name="s")
    run = pl.kernel(
        _gather_kernel,
        out_type=jax.ShapeDtypeStruct((TOTAL, EMBED_DIM), jnp.float32),
        mesh=mesh,
        scratch_types=[
            pltpu.VMEM((NCHUNKS, CHUNK), jnp.int32),
            pltpu.VMEM((CHUNK, EMBED_DIM), jnp.float32),
            pltpu.SemaphoreType.DMA,
            pltpu.SemaphoreType.DMA,
        ],
        compiler_params=pltpu.CompilerParams(use_tc_tiling_on_sc=False),
    )
    idx = ids_flat.reshape(NUM_WORKERS, NCHUNKS, CHUNK).astype(jnp.int32)
    return run(table, idx)


def kernel(champion_ids, table):
    out = _embed(champion_ids.reshape(-1), table)
    return out.reshape(BATCH, PER_TEAM, EMBED_DIM)


# speedup vs baseline: 6.2525x; 1.1992x over previous
import functools

import jax
import jax.numpy as jnp
from jax import lax
from jax.experimental import pallas as pl
from jax.experimental.pallas import tpu as pltpu
from jax.experimental.pallas import tpu_sc as plsc

NUM_CORES = 2
NUM_SUBCORES = 16
NUM_WORKERS = NUM_CORES * NUM_SUBCORES

BATCH = 16384
PER_TEAM = 50
EMBED_DIM = 64
TOTAL = BATCH * PER_TEAM
ROWS_PER_WORKER = TOTAL // NUM_WORKERS
CHUNK = 128
NCHUNKS = ROWS_PER_WORKER // CHUNK
NBUF = 4
NGROUPS = NCHUNKS // NBUF


def _gather_kernel(table_hbm, idx_hbm, out_hbm, idx_v, rows_v, gsems, wsems):
    wid = lax.axis_index("s") * NUM_CORES + lax.axis_index("c")
    base = wid * ROWS_PER_WORKER

    pltpu.sync_copy(idx_hbm.at[wid], idx_v)

    def fire_gather(c, b):
        pltpu.async_copy(table_hbm.at[idx_v.at[c]], rows_v.at[b], gsems.at[b])

    def fire_wb(c, b):
        pltpu.async_copy(rows_v.at[b],
                         out_hbm.at[pl.ds(base + c * CHUNK, CHUNK)], wsems.at[b])

    def wait_gather(b):
        pltpu.make_async_copy(out_hbm.at[pl.ds(0, CHUNK)], rows_v.at[b],
                              gsems.at[b]).wait()

    def wait_wb(b):
        pltpu.make_async_copy(rows_v.at[b], out_hbm.at[pl.ds(0, CHUNK)],
                              wsems.at[b]).wait()

    for b in range(NBUF):
        fire_gather(b, b)

    def group(g, carry):
        for b in range(NBUF):
            c = g * NBUF + b
            wait_gather(b)
            fire_wb(c, b)
            wait_wb(b)
            fire_gather(c + NBUF, b)
        return carry

    lax.fori_loop(0, NGROUPS - 1, group, 0, unroll=False)

    for b in range(NBUF):
        c = (NGROUPS - 1) * NBUF + b
        wait_gather(b)
        fire_wb(c, b)
    for b in range(NBUF):
        wait_wb(b)


@jax.jit
def _embed(ids_flat, table):
    mesh = plsc.VectorSubcoreMesh(core_axis_name="c", subcore_axis_name="s")
    run = pl.kernel(
        _gather_kernel,
        out_type=jax.ShapeDtypeStruct((TOTAL, EMBED_DIM), jnp.float32),
        mesh=mesh,
        scratch_types=[
            pltpu.VMEM((NCHUNKS, CHUNK), jnp.int32),
            pltpu.VMEM((NBUF, CHUNK, EMBED_DIM), jnp.float32),
            pltpu.SemaphoreType.DMA((NBUF,)),
            pltpu.SemaphoreType.DMA((NBUF,)),
        ],
        compiler_params=pltpu.CompilerParams(use_tc_tiling_on_sc=False),
    )
    idx = ids_flat.reshape(NUM_WORKERS, NCHUNKS, CHUNK).astype(jnp.int32)
    return run(table, idx)


def kernel(champion_ids, table):
    out = _embed(champion_ids.reshape(-1), table)
    return out.reshape(BATCH, PER_TEAM, EMBED_DIM)
